# baseline (speedup 1.0000x reference)
import functools

import jax
import jax.numpy as jnp
from jax import lax
from jax.experimental import pallas as pl
from jax.experimental.pallas import tpu as pltpu
from jax.experimental.pallas import tpu_sc as plsc

LANES = 16


@functools.lru_cache(maxsize=None)
def _build_word_gather(B2, HALF_V, BASE, D, NC, NS):
    NW = NC * NS
    ROWS_PER_W = B2 // NW
    K = 32
    NBATCH = ROWS_PER_W // K
    KD = D // LANES
    PAIRS_PER_BATCH = K // 2

    mesh = plsc.VectorSubcoreMesh(core_axis_name="c", subcore_axis_name="s")

    @functools.partial(
        pl.kernel,
        mesh=mesh,
        out_type=jax.ShapeDtypeStruct((B2 // 2, 2 * D), jnp.float32),
        scratch_types=[
            pltpu.VMEM((ROWS_PER_W,), jnp.int32),
            pltpu.VMEM((2, K, 8, D), jnp.float32),
            pltpu.VMEM((2, PAIRS_PER_BATCH, 2 * D), jnp.float32),
            pltpu.SemaphoreType.DMA,
            pltpu.SemaphoreType.DMA,
            pltpu.SemaphoreType.DMA,
            pltpu.SemaphoreType.DMA,
        ],
    )
    def w_kernel(widx_hbm, wtab_hbm, out_hbm,
                 widx_v, wtile_v, wout_v, gsem0, gsem1, osem0, osem1):
        gsem = (gsem0, gsem1)
        osem = (osem0, osem1)
        wid = lax.axis_index("s") * NC + lax.axis_index("c")
        row0 = wid * ROWS_PER_W

        pltpu.sync_copy(widx_hbm.at[pl.ds(row0, ROWS_PER_W)], widx_v)

        def fire_batch(bi, p):
            for h in range(K // LANES):
                iv = widx_v[pl.ds(bi * K + h * LANES, LANES)]
                t8v = jnp.clip(iv - BASE, 0, HALF_V - 8) & ~7
                for i in range(LANES):
                    t8 = pl.multiple_of(t8v[i], 8)
                    pltpu.async_copy(wtab_hbm.at[pl.ds(t8, 8)],
                                     wtile_v.at[p, h * LANES + i], gsem[p])

        def wait_batch(p):
            for i in range(K):
                pltpu.make_async_copy(wtab_hbm.at[pl.ds(0, 8)],
                                      wtile_v.at[p, i], gsem[p]).wait()

        def select_batch(bi, p):
            for h in range(K // LANES):
                remv = (widx_v[pl.ds(bi * K + h * LANES, LANES)] - BASE) & 7
                _select_group(bi, p, h, remv)

        def _select_group(bi, p, h, remv):
            for i16 in range(LANES):
                i = LANES * h + i16
                rem = remv[i16]
                for k in range(KD):
                    wout_v[p, i // 2, (i % 2) * D + k * LANES:
                           (i % 2) * D + (k + 1) * LANES] = (
                        wtile_v[p, i, rem, pl.ds(k * LANES, LANES)])

        def fire_out(bi, p):
            off = pl.multiple_of(row0 // 2 + bi * PAIRS_PER_BATCH, 8)
            pltpu.async_copy(wout_v.at[p],
                             out_hbm.at[pl.ds(off, PAIRS_PER_BATCH)], osem[p])

        def wait_out(p):
            pltpu.make_async_copy(
                wout_v.at[p],
                out_hbm.at[pl.ds(0, PAIRS_PER_BATCH)], osem[p]).wait()

        fire_batch(0, 0)

        def pair_body(b2, carry):
            for p in range(2):
                bi = b2 * 2 + p

                @pl.when(bi + 1 < NBATCH)
                def _():
                    fire_batch(bi + 1, 1 - p)

                wait_batch(p)

                @pl.when(bi >= 2)
                def _():
                    wait_out(p)

                select_batch(bi, p)
                fire_out(bi, p)
            return carry

        lax.fori_loop(0, NBATCH // 2, pair_body, 0, unroll=False)
        wait_out(0)
        wait_out(1)

    return w_kernel


@functools.lru_cache(maxsize=None)
def _build_subword_pool(B2, SV, D, NSUB, NC, NS):
    NW = NC * NS
    ROWS_PER_W = B2 // NW
    CHUNK = 32
    NCHUNK = ROWS_PER_W // CHUNK
    KD = D // LANES

    mesh = plsc.VectorSubcoreMesh(core_axis_name="c", subcore_axis_name="s")

    @functools.partial(
        pl.kernel,
        mesh=mesh,
        compiler_params=pltpu.CompilerParams(use_tc_tiling_on_sc=False),
        out_type=jax.ShapeDtypeStruct((B2, D), jnp.float32),
        scratch_types=[
            pltpu.VMEM((NSUB, ROWS_PER_W), jnp.int32),
            pltpu.VMEM((2, NSUB, CHUNK, D), jnp.float32),
            pltpu.VMEM((2, CHUNK, D), jnp.float32),
            pltpu.SemaphoreType.DMA,
            pltpu.SemaphoreType.DMA,
            pltpu.SemaphoreType.DMA,
            pltpu.SemaphoreType.DMA,
        ],
    )
    def s1_kernel(tsubT_hbm, osubT_hbm, stab_hbm, out_hbm,
                  sidx_v, srows_v, obuf_v, gsem0, gsem1, osem0, osem1):
        gsem = (gsem0, gsem1)
        osem = (osem0, osem1)
        wid = lax.axis_index("s") * NC + lax.axis_index("c")
        row0 = wid * ROWS_PER_W

        HALF = NW // 2

        @pl.when(wid < HALF)
        def _():
            pltpu.sync_copy(
                tsubT_hbm.at[:, pl.ds(pl.multiple_of(row0, 128),
                                      ROWS_PER_W)], sidx_v)

        @pl.when(wid >= HALF)
        def _():
            pltpu.sync_copy(
                osubT_hbm.at[:, pl.ds(pl.multiple_of(row0 - HALF *
                                                     ROWS_PER_W, 128),
                                      ROWS_PER_W)], sidx_v)

        def fire_gathers(g, b):
            for j in range(NSUB):
                pltpu.async_copy(
                    stab_hbm.at[sidx_v.at[j, pl.ds(g * CHUNK, CHUNK)]],
                    srows_v.at[b, j], gsem[b])

        def wait_gathers(b):
            for j in range(NSUB):
                pltpu.make_async_copy(
                    stab_hbm.at[sidx_v.at[j, pl.ds(0, CHUNK)]],
                    srows_v.at[b, j], gsem[b]).wait()

        def fire_out(g, b):
            base = pl.multiple_of(row0 + g * CHUNK, 8)
            pltpu.async_copy(obuf_v.at[b], out_hbm.at[pl.ds(base, CHUNK)],
                             osem[b])

        def wait_out(b):
            pltpu.make_async_copy(obuf_v.at[b],
                                  out_hbm.at[pl.ds(0, CHUNK)], osem[b]).wait()

        def compute(b):
            def row_body(r, carry2):
                for k in range(KD):
                    col = pl.ds(k * LANES, LANES)
                    acc = srows_v[b, 0, r, col]
                    for j in range(1, NSUB):
                        acc = acc + srows_v[b, j, r, col]
                    obuf_v[b, r, col] = acc
                return carry2

            lax.fori_loop(0, CHUNK, row_body, 0, unroll=False)

        fire_gathers(0, 0)

        def pair_body(g2, carry):
            for b in range(2):
                g = g2 * 2 + b

                @pl.when(g + 1 < NCHUNK)
                def _():
                    fire_gathers(g + 1, 1 - b)

                wait_gathers(b)

                @pl.when(g >= 2)
                def _():
                    wait_out(b)

                compute(b)
                fire_out(g, b)
            return carry

        lax.fori_loop(0, NCHUNK // 2, pair_body, 0, unroll=False)
        wait_out(0)
        wait_out(1)

    return s1_kernel


@functools.lru_cache(maxsize=None)
def _build_add(B2, D, NC, NS, HALF_V):
    NW = NC * NS
    ROWS_PER_W = B2 // NW
    CHUNK = 128
    NCHUNK = ROWS_PER_W // CHUNK
    CPAIR = CHUNK // 2
    KD = D // LANES

    mesh = plsc.VectorSubcoreMesh(core_axis_name="c", subcore_axis_name="s")

    @functools.partial(
        pl.kernel,
        mesh=mesh,
        compiler_params=pltpu.CompilerParams(use_tc_tiling_on_sc=False),
        out_type=jax.ShapeDtypeStruct((B2, D), jnp.float32),
        scratch_types=[
            pltpu.VMEM((2, CHUNK, D), jnp.float32),
            pltpu.VMEM((2, CPAIR, 2 * D), jnp.float32),
            pltpu.VMEM((2, CPAIR, 2 * D), jnp.float32),
            pltpu.VMEM((2, CHUNK), jnp.int32),
            pltpu.VMEM((2, CHUNK, D), jnp.float32),
            pltpu.SemaphoreType.DMA,
            pltpu.SemaphoreType.DMA,
            pltpu.SemaphoreType.DMA,
            pltpu.SemaphoreType.DMA,
        ],
    )
    def s2_kernel(sub_hbm, wpa_hbm, wpb_hbm, widx_hbm, out_hbm,
                  sub_v, wpa_v, wpb_v, idx_v, obuf_v,
                  gsem0, gsem1, osem0, osem1):
        gsem = (gsem0, gsem1)
        osem = (osem0, osem1)
        wid = lax.axis_index("s") * NC + lax.axis_index("c")
        row0 = wid * ROWS_PER_W

        def fire_in(g, b):
            base = pl.multiple_of(row0 + g * CHUNK, 8)
            pbase = pl.multiple_of(row0 // 2 + g * CPAIR, 8)
            pltpu.async_copy(sub_hbm.at[pl.ds(base, CHUNK)],
                             sub_v.at[b], gsem[b])
            pltpu.async_copy(wpa_hbm.at[pl.ds(pbase, CPAIR)],
                             wpa_v.at[b], gsem[b])
            pltpu.async_copy(wpb_hbm.at[pl.ds(pbase, CPAIR)],
                             wpb_v.at[b], gsem[b])
            pltpu.async_copy(widx_hbm.at[pl.ds(base, CHUNK)],
                             idx_v.at[b], gsem[b])

        def wait_in(b):
            pltpu.make_async_copy(sub_hbm.at[pl.ds(0, CHUNK)],
                                  sub_v.at[b], gsem[b]).wait()
            pltpu.make_async_copy(wpa_hbm.at[pl.ds(0, CPAIR)],
                                  wpa_v.at[b], gsem[b]).wait()
            pltpu.make_async_copy(wpb_hbm.at[pl.ds(0, CPAIR)],
                                  wpb_v.at[b], gsem[b]).wait()
            pltpu.make_async_copy(widx_hbm.at[pl.ds(0, CHUNK)],
                                  idx_v.at[b], gsem[b]).wait()

        def fire_out(g, b):
            base = pl.multiple_of(row0 + g * CHUNK, 8)
            pltpu.async_copy(obuf_v.at[b], out_hbm.at[pl.ds(base, CHUNK)],
                             osem[b])

        def wait_out(b):
            pltpu.make_async_copy(obuf_v.at[b],
                                  out_hbm.at[pl.ds(0, CHUNK)], osem[b]).wait()

        def compute(b):
            def group_rows(g16, carry2):
                idxv = idx_v[b, pl.ds(g16 * LANES, LANES)]
                for i16 in range(LANES):
                    cond = idxv[i16] < HALF_V
                    q = g16 * (LANES // 2) + i16 // 2
                    half = i16 % 2
                    r = g16 * LANES + i16
                    for k in range(KD):
                        col = pl.ds(k * LANES, LANES)
                        pcol = pl.ds(half * D + k * LANES, LANES)
                        wrow = jnp.where(cond, wpa_v[b, q, pcol],
                                         wpb_v[b, q, pcol])
                        obuf_v[b, r, col] = sub_v[b, r, col] + wrow
                return carry2

            lax.fori_loop(0, CHUNK // LANES, group_rows, 0, unroll=False)

        fire_in(0, 0)

        def pair_body(g2, carry):
            for b in range(2):
                g = g2 * 2 + b

                @pl.when(g + 1 < NCHUNK)
                def _():
                    fire_in(g + 1, 1 - b)

                wait_in(b)

                @pl.when(g >= 2)
                def _():
                    wait_out(b)

                compute(b)
                fire_out(g, b)
            return carry

        lax.fori_loop(0, NCHUNK // 2, pair_body, 0, unroll=False)
        wait_out(0)
        wait_out(1)

    return s2_kernel


def kernel(target, other, target_sub, other_sub, word_embed, subword_embed):
    B = target.shape[0]
    NSUB = target_sub.shape[1]
    V, D = word_embed.shape
    SV = subword_embed.shape[0]
    info = plsc.get_sparse_core_info()
    NC, NS = info.num_cores, info.num_subcores

    half_v = V // 2
    wa_kernel = _build_word_gather(2 * B, half_v, 0, D, NC, NS)
    wb_kernel = _build_word_gather(2 * B, half_v, half_v, D, NC, NS)
    s1_kernel = _build_subword_pool(2 * B, SV, D, NSUB, NC, NS)
    s2_kernel = _build_add(2 * B, D, NC, NS, half_v)

    widx = jnp.concatenate([target, other]).astype(jnp.int32)
    subout = s1_kernel(target_sub.T.astype(jnp.int32),
                       other_sub.T.astype(jnp.int32), subword_embed)
    wpa = wa_kernel(widx, word_embed[:half_v])
    wpb = wb_kernel(widx, word_embed[half_v:])
    out = s2_kernel(subout, wpa, wpb, widx)
    return out[:B], out[B:]


# speedup vs baseline: 3.5617x; 3.5617x over previous
import functools

import jax
import jax.numpy as jnp
from jax import lax
from jax.experimental import pallas as pl
from jax.experimental.pallas import tpu as pltpu
from jax.experimental.pallas import tpu_sc as plsc

LANES = 16


@functools.lru_cache(maxsize=None)
def _build_word_gather(B2, V, D, NC, NS):
    NW = NC * NS
    ROWS_PER_W = B2 // NW
    K = 32
    NBATCH = ROWS_PER_W // K
    KD = D // LANES
    PAIRS_PER_BATCH = K // 2

    mesh = plsc.VectorSubcoreMesh(core_axis_name="c", subcore_axis_name="s")

    @functools.partial(
        pl.kernel,
        mesh=mesh,
        out_type=jax.ShapeDtypeStruct((B2 // 2, 2 * D), jnp.float32),
        scratch_types=[
            pltpu.VMEM((ROWS_PER_W,), jnp.int32),
            pltpu.VMEM((2, K, 8, D), jnp.float32),
            pltpu.VMEM((2, PAIRS_PER_BATCH, 2 * D), jnp.float32),
            pltpu.SemaphoreType.DMA,
            pltpu.SemaphoreType.DMA,
            pltpu.SemaphoreType.DMA,
            pltpu.SemaphoreType.DMA,
        ],
    )
    def w_kernel(widx_hbm, wtab_hbm, out_hbm,
                 widx_v, wtile_v, wout_v, gsem0, gsem1, osem0, osem1):
        gsem = (gsem0, gsem1)
        osem = (osem0, osem1)
        wid = lax.axis_index("s") * NC + lax.axis_index("c")
        row0 = wid * ROWS_PER_W

        pltpu.sync_copy(widx_hbm.at[pl.ds(row0, ROWS_PER_W)], widx_v)

        def fire_batch(bi, p):
            for h in range(K // LANES):
                iv = widx_v[pl.ds(bi * K + h * LANES, LANES)]
                t8v = iv & ~7
                for i in range(LANES):
                    t8 = pl.multiple_of(t8v[i], 8)
                    pltpu.async_copy(wtab_hbm.at[pl.ds(t8, 8)],
                                     wtile_v.at[p, h * LANES + i], gsem[p])

        def wait_batch(p):
            for i in range(K):
                pltpu.make_async_copy(wtab_hbm.at[pl.ds(0, 8)],
                                      wtile_v.at[p, i], gsem[p]).wait()

        def select_batch(bi, p):
            for h in range(K // LANES):
                remv = widx_v[pl.ds(bi * K + h * LANES, LANES)] & 7
                _select_group(bi, p, h, remv)

        def _select_group(bi, p, h, remv):
            for i16 in range(LANES):
                i = LANES * h + i16
                rem = remv[i16]
                for k in range(KD):
                    wout_v[p, i // 2, (i % 2) * D + k * LANES:
                           (i % 2) * D + (k + 1) * LANES] = (
                        wtile_v[p, i, rem, pl.ds(k * LANES, LANES)])

        def fire_out(bi, p):
            off = pl.multiple_of(row0 // 2 + bi * PAIRS_PER_BATCH, 8)
            pltpu.async_copy(wout_v.at[p],
                             out_hbm.at[pl.ds(off, PAIRS_PER_BATCH)], osem[p])

        def wait_out(p):
            pltpu.make_async_copy(
                wout_v.at[p],
                out_hbm.at[pl.ds(0, PAIRS_PER_BATCH)], osem[p]).wait()

        fire_batch(0, 0)

        def pair_body(b2, carry):
            for p in range(2):
                bi = b2 * 2 + p

                @pl.when(bi + 1 < NBATCH)
                def _():
                    fire_batch(bi + 1, 1 - p)

                wait_batch(p)

                @pl.when(bi >= 2)
                def _():
                    wait_out(p)

                select_batch(bi, p)
                fire_out(bi, p)
            return carry

        lax.fori_loop(0, NBATCH // 2, pair_body, 0, unroll=False)
        wait_out(0)
        wait_out(1)

    return w_kernel


@functools.lru_cache(maxsize=None)
def _build_subword_pool(B2, SV, D, NSUB, NC, NS):
    NW = NC * NS
    ROWS_PER_W = B2 // NW
    CHUNK = 32
    NCHUNK = ROWS_PER_W // CHUNK
    KD = D // LANES

    mesh = plsc.VectorSubcoreMesh(core_axis_name="c", subcore_axis_name="s")

    @functools.partial(
        pl.kernel,
        mesh=mesh,
        compiler_params=pltpu.CompilerParams(use_tc_tiling_on_sc=False),
        out_type=jax.ShapeDtypeStruct((B2, D), jnp.float32),
        scratch_types=[
            pltpu.VMEM((NSUB, ROWS_PER_W), jnp.int32),
            pltpu.VMEM((2, NSUB, CHUNK, D), jnp.float32),
            pltpu.VMEM((2, CHUNK, D), jnp.float32),
            pltpu.SemaphoreType.DMA,
            pltpu.SemaphoreType.DMA,
            pltpu.SemaphoreType.DMA,
            pltpu.SemaphoreType.DMA,
        ],
    )
    def s1_kernel(tsubT_hbm, osubT_hbm, stab_hbm, out_hbm,
                  sidx_v, srows_v, obuf_v, gsem0, gsem1, osem0, osem1):
        gsem = (gsem0, gsem1)
        osem = (osem0, osem1)
        wid = lax.axis_index("s") * NC + lax.axis_index("c")
        row0 = wid * ROWS_PER_W

        HALF = NW // 2

        @pl.when(wid < HALF)
        def _():
            pltpu.sync_copy(
                tsubT_hbm.at[:, pl.ds(pl.multiple_of(row0, 128),
                                      ROWS_PER_W)], sidx_v)

        @pl.when(wid >= HALF)
        def _():
            pltpu.sync_copy(
                osubT_hbm.at[:, pl.ds(pl.multiple_of(row0 - HALF *
                                                     ROWS_PER_W, 128),
                                      ROWS_PER_W)], sidx_v)

        def fire_gathers(g, b):
            for j in range(NSUB):
                pltpu.async_copy(
                    stab_hbm.at[sidx_v.at[j, pl.ds(g * CHUNK, CHUNK)]],
                    srows_v.at[b, j], gsem[b])

        def wait_gathers(b):
            for j in range(NSUB):
                pltpu.make_async_copy(
                    stab_hbm.at[sidx_v.at[j, pl.ds(0, CHUNK)]],
                    srows_v.at[b, j], gsem[b]).wait()

        def fire_out(g, b):
            base = pl.multiple_of(row0 + g * CHUNK, 8)
            pltpu.async_copy(obuf_v.at[b], out_hbm.at[pl.ds(base, CHUNK)],
                             osem[b])

        def wait_out(b):
            pltpu.make_async_copy(obuf_v.at[b],
                                  out_hbm.at[pl.ds(0, CHUNK)], osem[b]).wait()

        def compute(b):
            def row_body(r, carry2):
                for k in range(KD):
                    col = pl.ds(k * LANES, LANES)
                    acc = srows_v[b, 0, r, col]
                    for j in range(1, NSUB):
                        acc = acc + srows_v[b, j, r, col]
                    obuf_v[b, r, col] = acc
                return carry2

            lax.fori_loop(0, CHUNK, row_body, 0, unroll=False)

        fire_gathers(0, 0)

        def pair_body(g2, carry):
            for b in range(2):
                g = g2 * 2 + b

                @pl.when(g + 1 < NCHUNK)
                def _():
                    fire_gathers(g + 1, 1 - b)

                wait_gathers(b)

                @pl.when(g >= 2)
                def _():
                    wait_out(b)

                compute(b)
                fire_out(g, b)
            return carry

        lax.fori_loop(0, NCHUNK // 2, pair_body, 0, unroll=False)
        wait_out(0)
        wait_out(1)

    return s1_kernel


@functools.lru_cache(maxsize=None)
def _build_add(B2, D, NC, NS):
    NW = NC * NS
    ROWS_PER_W = B2 // NW
    CHUNK = 128
    NCHUNK = ROWS_PER_W // CHUNK
    CPAIR = CHUNK // 2
    KD = D // LANES

    mesh = plsc.VectorSubcoreMesh(core_axis_name="c", subcore_axis_name="s")

    @functools.partial(
        pl.kernel,
        mesh=mesh,
        compiler_params=pltpu.CompilerParams(use_tc_tiling_on_sc=False),
        out_type=jax.ShapeDtypeStruct((B2, D), jnp.float32),
        scratch_types=[
            pltpu.VMEM((2, CHUNK, D), jnp.float32),
            pltpu.VMEM((2, CPAIR, 2 * D), jnp.float32),
            pltpu.VMEM((2, CHUNK, D), jnp.float32),
            pltpu.SemaphoreType.DMA,
            pltpu.SemaphoreType.DMA,
            pltpu.SemaphoreType.DMA,
            pltpu.SemaphoreType.DMA,
        ],
    )
    def s2_kernel(sub_hbm, wpair_hbm, out_hbm,
                  sub_v, wp_v, obuf_v, gsem0, gsem1, osem0, osem1):
        gsem = (gsem0, gsem1)
        osem = (osem0, osem1)
        wid = lax.axis_index("s") * NC + lax.axis_index("c")
        row0 = wid * ROWS_PER_W

        def fire_in(g, b):
            base = pl.multiple_of(row0 + g * CHUNK, 8)
            pbase = pl.multiple_of(row0 // 2 + g * CPAIR, 8)
            pltpu.async_copy(sub_hbm.at[pl.ds(base, CHUNK)],
                             sub_v.at[b], gsem[b])
            pltpu.async_copy(wpair_hbm.at[pl.ds(pbase, CPAIR)],
                             wp_v.at[b], gsem[b])

        def wait_in(b):
            pltpu.make_async_copy(sub_hbm.at[pl.ds(0, CHUNK)],
                                  sub_v.at[b], gsem[b]).wait()
            pltpu.make_async_copy(wpair_hbm.at[pl.ds(0, CPAIR)],
                                  wp_v.at[b], gsem[b]).wait()

        def fire_out(g, b):
            base = pl.multiple_of(row0 + g * CHUNK, 8)
            pltpu.async_copy(obuf_v.at[b], out_hbm.at[pl.ds(base, CHUNK)],
                             osem[b])

        def wait_out(b):
            pltpu.make_async_copy(obuf_v.at[b],
                                  out_hbm.at[pl.ds(0, CHUNK)], osem[b]).wait()

        def compute(b):
            def pair_rows(q, carry2):
                for half in range(2):
                    for k in range(KD):
                        col = pl.ds(k * LANES, LANES)
                        pcol = pl.ds(half * D + k * LANES, LANES)
                        obuf_v[b, 2 * q + half, col] = (
                            sub_v[b, 2 * q + half, col] + wp_v[b, q, pcol])
                return carry2

            lax.fori_loop(0, CPAIR, pair_rows, 0, unroll=False)

        fire_in(0, 0)

        def pair_body(g2, carry):
            for b in range(2):
                g = g2 * 2 + b

                @pl.when(g + 1 < NCHUNK)
                def _():
                    fire_in(g + 1, 1 - b)

                wait_in(b)

                @pl.when(g >= 2)
                def _():
                    wait_out(b)

                compute(b)
                fire_out(g, b)
            return carry

        lax.fori_loop(0, NCHUNK // 2, pair_body, 0, unroll=False)
        wait_out(0)
        wait_out(1)

    return s2_kernel


def kernel(target, other, target_sub, other_sub, word_embed, subword_embed):
    B = target.shape[0]
    NSUB = target_sub.shape[1]
    V, D = word_embed.shape
    SV = subword_embed.shape[0]
    info = plsc.get_sparse_core_info()
    NC, NS = info.num_cores, info.num_subcores

    w_kernel = _build_word_gather(2 * B, V, D, NC, NS)
    s1_kernel = _build_subword_pool(2 * B, SV, D, NSUB, NC, NS)
    s2_kernel = _build_add(2 * B, D, NC, NS)

    widx = jnp.concatenate([target, other]).astype(jnp.int32)
    subout = s1_kernel(target_sub.T.astype(jnp.int32),
                       other_sub.T.astype(jnp.int32), subword_embed)
    wpair = w_kernel(widx, word_embed)
    out = s2_kernel(subout, wpair)
    return out[:B], out[B:]
